# baseline (speedup 1.0000x reference)
import jax
import jax.numpy as jnp
from jax.experimental import pallas as pl

BATCH = 1024
EMB = 512
NUM_CLASSES = 100000
TILE_N = 2048


def _mm_kernel(x_ref, w_ref, o_ref):
    w = w_ref[...].astype(jnp.bfloat16)
    o_ref[...] = jax.lax.dot_general(
        x_ref[...],
        w,
        dimension_numbers=(((1,), (1,)), ((), ())),
        preferred_element_type=jnp.float32,
    )


def kernel(total_features, norm_weight):
    x = total_features.astype(jnp.bfloat16)
    grid = (pl.cdiv(NUM_CLASSES, TILE_N),)
    return pl.pallas_call(
        _mm_kernel,
        grid=grid,
        in_specs=[
            pl.BlockSpec((BATCH, EMB), lambda i: (0, 0)),
            pl.BlockSpec((TILE_N, EMB), lambda i: (i, 0)),
        ],
        out_specs=pl.BlockSpec((BATCH, TILE_N), lambda i: (0, i)),
        out_shape=jax.ShapeDtypeStruct((BATCH, NUM_CLASSES), jnp.float32),
    )(x, norm_weight)
